# baseline (speedup 1.0000x reference)
import functools

import jax
import jax.numpy as jnp
from jax import lax
from jax.experimental import pallas as pl
from jax.experimental.pallas import tpu as pltpu
from jax.experimental.pallas import tpu_sc as plsc

NC = 2
NS = 16
NW = NC * NS
CHUNK = 128
HEADS = 3
EMB = 16
NBUF = 2
NIDX = 4
STEP = 4


def _prep_body(x_ref, wm_ref, bm_ref, wa_ref, d_ref, a_ref):
    xb = x_ref[...]
    hcols = []
    acols = []
    bcols = []
    for h in range(HEADS):
        hh = jnp.dot(xb, wm_ref[h], preferred_element_type=jnp.float32)
        hh = hh + bm_ref[h][None, :]
        hcols.append(hh)
        wa = wa_ref[h]
        acols.append(jnp.dot(hh, wa[0:EMB], preferred_element_type=jnp.float32))
        bcols.append(jnp.dot(hh, wa[EMB:2 * EMB], preferred_element_type=jnp.float32))
    z13 = jnp.zeros((xb.shape[0], 13), jnp.float32)
    d_ref[...] = jnp.concatenate(hcols + bcols + [z13], axis=1)
    a_ref[...] = jnp.concatenate(acols + [z13], axis=1)


def _fin_body(p_ref, o_ref):
    s = p_ref[0] + p_ref[1]
    parts = []
    for h in range(HEADS):
        den = s[:, 48 + h:49 + h]
        parts.append(s[:, h * EMB:(h + 1) * EMB] / den)
    o_ref[...] = jnp.concatenate(parts, axis=1)


LROWS = 512


def _make_sc_edge(NP, CPW):
    mesh = plsc.VectorSubcoreMesh(
        core_axis_name="c", subcore_axis_name="s", num_cores=NC, num_subcores=NS)
    ACCR = NP + ((2 * LROWS + NS * CHUNK - 1) // (NS * CHUNK)) * NS * CHUNK
    rptz = ACCR // NS
    nblkz = rptz // CHUNK
    rpt = NP // NS
    nblk = rpt // CHUNK

    @functools.partial(
        pl.kernel,
        out_type=jax.ShapeDtypeStruct((NC, NP, 64), jnp.float32),
        mesh=mesh,
        compiler_params=pltpu.CompilerParams(use_tc_tiling_on_sc=False),
        scratch_types=[
            pltpu.VMEM((NIDX, CHUNK), jnp.int32),
            pltpu.VMEM((NIDX, CHUNK), jnp.int32),
            pltpu.VMEM((NBUF, CHUNK, 16), jnp.float32),
            pltpu.VMEM((NBUF, CHUNK, 64), jnp.float32),
            pltpu.VMEM((LROWS, 64), jnp.float32),
            pltpu.VMEM((LROWS // CHUNK, CHUNK), jnp.int32),
            pltpu.VMEM_SHARED((ACCR, 64), jnp.float32),
            [pltpu.SemaphoreType.DMA] * NIDX,
            [pltpu.SemaphoreType.DMA] * NBUF,
        ],
    )
    def sc_edge(dtab, atab, srcp, dstp, out, srcq, dstq, arows, drows,
                accl, flidx, acc_sh, semi, semg):
        cid = lax.axis_index("c")
        sid = lax.axis_index("s")
        wid = sid * NC + cid

        def stage(n, q, fire):
            base = pl.multiple_of((wid * CPW + n) * CHUNK, CHUNK)
            d1 = pltpu.make_async_copy(srcp.at[pl.ds(base, CHUNK)],
                                       srcq.at[q], semi[q])
            d2 = pltpu.make_async_copy(dstp.at[pl.ds(base, CHUNK)],
                                       dstq.at[q], semi[q])
            if fire:
                d1.start()
                d2.start()
            else:
                d1.wait()
                d2.wait()

        def gathers(q, p, fire):
            d1 = pltpu.make_async_copy(atab.at[srcq.at[q]], arows.at[p], semg[p])
            d2 = pltpu.make_async_copy(dtab.at[dstq.at[q]], drows.at[p], semg[p])
            if fire:
                d1.start()
                d2.start()
            else:
                d1.wait()
                d2.wait()

        def zrow(r, carry):
            z16 = jnp.zeros((16,), jnp.float32)
            for j in range(4):
                accl[r, pl.ds(j * 16, 16)] = z16
            return carry
        lax.fori_loop(0, LROWS, zrow, 0)
        for t in range(nblkz):
            dst_rows = pl.ds(pl.multiple_of(sid * rptz + t * CHUNK, CHUNK), CHUNK)
            pltpu.sync_copy(accl.at[pl.ds(0, CHUNK)], acc_sh.at[dst_rows])
        plsc.subcore_barrier()

        lane = lax.iota(jnp.int32, 16)

        def compute(p, q, node_base):
            def cgrp(i, c2):
                base2 = pl.multiple_of(i * 16, 16)
                s16 = jnp.minimum(srcq[q, pl.ds(base2, 16)] - node_base,
                                  LROWS - 1)
                for j in range(16):
                    e = base2 + j
                    t = arows[p, e, :] + drows[p, e, pl.ds(48, 16)]
                    t = jnp.where(t >= 0.0, t, 0.2 * t)
                    t = jnp.clip(t, -2.0, 2.0)
                    sv = jnp.exp(t)
                    sj = s16[j]
                    plsc.addupdate(accl.at[sj, pl.ds(48, 16)],
                                   jnp.where(lane < HEADS, sv, 0.0))
                    for h in range(HEADS):
                        plsc.addupdate(accl.at[sj, pl.ds(h * EMB, 16)],
                                       drows[p, e, pl.ds(h * EMB, 16)] * sv[h])
                return c2
            lax.fori_loop(0, CHUNK // 16, cgrp, 0)

        for q in range(NIDX):
            stage(q, q, True)
        stage(0, 0, False)
        gathers(0, 0, True)
        node_base = srcq[0, pl.ds(0, 16)][0]

        def super_body(g, carry):
            n0 = g * STEP
            for b in range(STEP):
                n = n0 + b
                stage(n + 1, (b + 1) % NIDX, False)
                gathers((b + 1) % NIDX, (b + 1) % NBUF, True)
                gathers(b % NIDX, b % NBUF, False)
                compute(b % NBUF, b % NIDX, node_base)
                stage(n + NIDX, b % NIDX, True)
            return carry
        lax.fori_loop(0, CPW // STEP, super_body, 0)

        gathers(CPW % NIDX, CPW % NBUF, False)
        for k in range(1, NIDX):
            stage(CPW + k, (CPW + k) % NIDX, False)

        def fxi(i, carry):
            base2 = pl.multiple_of(i * 16, 16)
            v = lane + (node_base + base2)
            for r in range(LROWS // CHUNK):
                flidx[r, pl.ds(base2, 16)] = v + r * CHUNK
            return carry
        lax.fori_loop(0, CHUNK // 16, fxi, 0)
        for r in range(LROWS // CHUNK):
            pltpu.sync_copy(accl.at[pl.ds(pl.multiple_of(r * CHUNK, CHUNK), CHUNK)],
                            acc_sh.at[flidx.at[r]], add=True)

        plsc.subcore_barrier()
        for t in range(nblk):
            rows = pl.ds(pl.multiple_of(sid * rpt + t * CHUNK, CHUNK), CHUNK)
            pltpu.sync_copy(acc_sh.at[rows], drows.at[0])
            pltpu.sync_copy(drows.at[0], out.at[cid, rows])

    return sc_edge


def kernel(node_features, A_tilde, edges, W_mlp, b_mlp, W_attn):
    x = node_features[0]
    N, _ = x.shape
    E = edges.shape[0]
    R = 1024
    NP = ((N + R - 1) // R) * R

    xp = jnp.pad(x, ((0, NP - N), (0, 0)))
    grid = NP // R
    d_tab, a_tab = pl.pallas_call(
        _prep_body,
        grid=(grid,),
        in_specs=[
            pl.BlockSpec((R, 128), lambda i: (i, 0)),
            pl.BlockSpec((HEADS, 128, EMB), lambda i: (0, 0, 0)),
            pl.BlockSpec((HEADS, EMB), lambda i: (0, 0)),
            pl.BlockSpec((HEADS, 2 * EMB, 1), lambda i: (0, 0, 0)),
        ],
        out_specs=[
            pl.BlockSpec((R, 64), lambda i: (i, 0)),
            pl.BlockSpec((R, 16), lambda i: (i, 0)),
        ],
        out_shape=[
            jax.ShapeDtypeStruct((NP, 64), jnp.float32),
            jax.ShapeDtypeStruct((NP, 16), jnp.float32),
        ],
    )(xp, W_mlp, b_mlp, W_attn)

    src = edges[:, 0].astype(jnp.int32)
    dst = edges[:, 1].astype(jnp.int32)
    CPW = -(-E // (NW * CHUNK))
    CPW = ((CPW + STEP - 1) // STEP) * STEP
    EP = NW * CPW * CHUNK
    pad = jnp.full((EP - E + (NIDX + 1) * CHUNK,), N, jnp.int32)
    srcp = jnp.concatenate([src, pad])
    dstp = jnp.concatenate([dst, pad])

    partial = _make_sc_edge(NP, CPW)(d_tab, a_tab, srcp, dstp)

    out48 = pl.pallas_call(
        _fin_body,
        grid=(grid,),
        in_specs=[pl.BlockSpec((NC, R, 64), lambda i: (0, i, 0))],
        out_specs=pl.BlockSpec((R, HEADS * EMB), lambda i: (i, 0)),
        out_shape=jax.ShapeDtypeStruct((NP, HEADS * EMB), jnp.float32),
    )(partial)

    return out48[None, :N, :]


# speedup vs baseline: 1.0184x; 1.0184x over previous
import functools

import jax
import jax.numpy as jnp
from jax import lax
from jax.experimental import pallas as pl
from jax.experimental.pallas import tpu as pltpu
from jax.experimental.pallas import tpu_sc as plsc

NC = 2
NS = 16
NW = NC * NS
CHUNK = 128
HEADS = 3
EMB = 16
NBUF = 2
NIDX = 4
STEP = 4


def _prep_body(x_ref, wm_ref, bm_ref, wa_ref, d_ref, a_ref):
    xb = x_ref[...]
    hcols = []
    acols = []
    bcols = []
    for h in range(HEADS):
        hh = jnp.dot(xb, wm_ref[h], preferred_element_type=jnp.float32)
        hh = hh + bm_ref[h][None, :]
        hcols.append(hh)
        wa = wa_ref[h]
        acols.append(jnp.dot(hh, wa[0:EMB], preferred_element_type=jnp.float32))
        bcols.append(jnp.dot(hh, wa[EMB:2 * EMB], preferred_element_type=jnp.float32))
    z13 = jnp.zeros((xb.shape[0], 13), jnp.float32)
    d_ref[...] = jnp.concatenate(hcols + bcols + [z13], axis=1)
    a_ref[...] = jnp.concatenate(acols + [z13], axis=1)


def _fin_body(p_ref, o_ref):
    s = p_ref[0] + p_ref[1]
    parts = []
    for h in range(HEADS):
        den = s[:, 48 + h:49 + h]
        parts.append(s[:, h * EMB:(h + 1) * EMB] / den)
    o_ref[...] = jnp.concatenate(parts, axis=1)


def _make_sc_edge(NP, CPW):
    mesh = plsc.VectorSubcoreMesh(
        core_axis_name="c", subcore_axis_name="s", num_cores=NC, num_subcores=NS)
    rpt = NP // NS
    nblk = rpt // CHUNK

    @functools.partial(
        pl.kernel,
        out_type=jax.ShapeDtypeStruct((NC, NP, 64), jnp.float32),
        mesh=mesh,
        compiler_params=pltpu.CompilerParams(use_tc_tiling_on_sc=False),
        scratch_types=[
            pltpu.VMEM((NIDX, CHUNK), jnp.int32),
            pltpu.VMEM((NIDX, CHUNK), jnp.int32),
            pltpu.VMEM((NBUF, CHUNK, 16), jnp.float32),
            pltpu.VMEM((NBUF, CHUNK, 64), jnp.float32),
            pltpu.VMEM((CHUNK, 64), jnp.float32),
            pltpu.VMEM_SHARED((NP, 64), jnp.float32),
            [pltpu.SemaphoreType.DMA] * NIDX,
            [pltpu.SemaphoreType.DMA] * NBUF,
        ],
    )
    def sc_edge(dtab, atab, srcp, dstp, out, srcq, dstq, arows, drows,
                contrib_v, acc_sh, semi, semg):
        cid = lax.axis_index("c")
        sid = lax.axis_index("s")
        wid = sid * NC + cid

        def stage(n, q, fire):
            base = pl.multiple_of((wid * CPW + n) * CHUNK, CHUNK)
            d1 = pltpu.make_async_copy(srcp.at[pl.ds(base, CHUNK)],
                                       srcq.at[q], semi[q])
            d2 = pltpu.make_async_copy(dstp.at[pl.ds(base, CHUNK)],
                                       dstq.at[q], semi[q])
            if fire:
                d1.start()
                d2.start()
            else:
                d1.wait()
                d2.wait()

        def gathers(q, p, fire):
            d1 = pltpu.make_async_copy(atab.at[srcq.at[q]], arows.at[p], semg[p])
            d2 = pltpu.make_async_copy(dtab.at[dstq.at[q]], drows.at[p], semg[p])
            if fire:
                d1.start()
                d2.start()
            else:
                d1.wait()
                d2.wait()

        def zrow(r, carry):
            z16 = jnp.zeros((16,), jnp.float32)
            for j in range(4):
                contrib_v[r, pl.ds(j * 16, 16)] = z16
            return carry
        lax.fori_loop(0, CHUNK, zrow, 0)
        for t in range(nblk):
            dst_rows = pl.ds(pl.multiple_of(sid * rpt + t * CHUNK, CHUNK), CHUNK)
            pltpu.sync_copy(contrib_v, acc_sh.at[dst_rows])
        plsc.subcore_barrier()

        lane = lax.iota(jnp.int32, 16)

        def compute(p, q):
            def cgrp(i, c2):
                base2 = pl.multiple_of(i * 16, 16)
                for j in range(16):
                    e = base2 + j
                    t = arows[p, e, :] + drows[p, e, pl.ds(48, 16)]
                    t = jnp.where(t >= 0.0, t, 0.2 * t)
                    t = jnp.clip(t, -2.0, 2.0)
                    sv = jnp.exp(t)
                    contrib_v[e, pl.ds(48, 16)] = jnp.where(lane < HEADS, sv, 0.0)
                    for h in range(HEADS):
                        contrib_v[e, pl.ds(h * EMB, 16)] = \
                            drows[p, e, pl.ds(h * EMB, 16)] * sv[h]
                return c2
            lax.fori_loop(0, CHUNK // 16, cgrp, 0)
            pltpu.sync_copy(contrib_v, acc_sh.at[srcq.at[q]], add=True)

        for q in range(NIDX):
            stage(q, q, True)
        stage(0, 0, False)
        gathers(0, 0, True)

        def super_body(g, carry):
            n0 = g * STEP
            for b in range(STEP):
                n = n0 + b
                stage(n + 1, (b + 1) % NIDX, False)
                gathers((b + 1) % NIDX, (b + 1) % NBUF, True)
                gathers(b % NIDX, b % NBUF, False)
                compute(b % NBUF, b % NIDX)
                stage(n + NIDX, b % NIDX, True)
            return carry
        lax.fori_loop(0, CPW // STEP, super_body, 0)

        gathers(CPW % NIDX, CPW % NBUF, False)
        for k in range(1, NIDX):
            stage(CPW + k, (CPW + k) % NIDX, False)

        plsc.subcore_barrier()
        for t in range(nblk):
            rows = pl.ds(pl.multiple_of(sid * rpt + t * CHUNK, CHUNK), CHUNK)
            pltpu.sync_copy(acc_sh.at[rows], contrib_v)
            pltpu.sync_copy(contrib_v, out.at[cid, rows])

    return sc_edge


def kernel(node_features, A_tilde, edges, W_mlp, b_mlp, W_attn):
    x = node_features[0]
    N, _ = x.shape
    E = edges.shape[0]
    R = 1024
    NP = ((N + R - 1) // R) * R

    xp = jnp.pad(x, ((0, NP - N), (0, 0)))
    grid = NP // R
    d_tab, a_tab = pl.pallas_call(
        _prep_body,
        grid=(grid,),
        in_specs=[
            pl.BlockSpec((R, 128), lambda i: (i, 0)),
            pl.BlockSpec((HEADS, 128, EMB), lambda i: (0, 0, 0)),
            pl.BlockSpec((HEADS, EMB), lambda i: (0, 0)),
            pl.BlockSpec((HEADS, 2 * EMB, 1), lambda i: (0, 0, 0)),
        ],
        out_specs=[
            pl.BlockSpec((R, 64), lambda i: (i, 0)),
            pl.BlockSpec((R, 16), lambda i: (i, 0)),
        ],
        out_shape=[
            jax.ShapeDtypeStruct((NP, 64), jnp.float32),
            jax.ShapeDtypeStruct((NP, 16), jnp.float32),
        ],
    )(xp, W_mlp, b_mlp, W_attn)

    src = edges[:, 0].astype(jnp.int32)
    dst = edges[:, 1].astype(jnp.int32)
    CPW = -(-E // (NW * CHUNK))
    CPW = ((CPW + STEP - 1) // STEP) * STEP
    EP = NW * CPW * CHUNK
    pad = jnp.full((EP - E + (NIDX + 1) * CHUNK,), N, jnp.int32)
    srcp = jnp.concatenate([src, pad])
    dstp = jnp.concatenate([dst, pad])

    partial = _make_sc_edge(NP, CPW)(d_tab, a_tab, srcp, dstp)

    out48 = pl.pallas_call(
        _fin_body,
        grid=(grid,),
        in_specs=[pl.BlockSpec((NC, R, 64), lambda i: (0, i, 0))],
        out_specs=pl.BlockSpec((R, HEADS * EMB), lambda i: (i, 0)),
        out_shape=jax.ShapeDtypeStruct((NP, HEADS * EMB), jnp.float32),
    )(partial)

    return out48[None, :N, :]
